# Initial kernel scaffold; baseline (speedup 1.0000x reference)
import jax
import jax.numpy as jnp
from jax.experimental import pallas as pl


def kernel(node_features, edge_features, us, vs, mol_node_matrix, mol_node_mask, node_edge_matrix, node_edge_mask, global_mask, W_n, b_n, W_e, b_e, Wm, bm, Wa, ba, Wen, ben, Wih, bih, Whh, bhh, Wra, bra, Wro, bro):
    raise NotImplementedError("write your pallas kernel here")


# speedup vs baseline: 2.6673x; 2.6673x over previous
import functools

import jax
import jax.numpy as jnp
from jax import lax
from jax.experimental import pallas as pl
from jax.experimental.pallas import tpu as pltpu
from jax.experimental.pallas import tpu_sc as plsc

N = 2048
E = 8192
M = 128
N_DIM = 64
E_DIM = 16
H = 128
HE = 64
L = 3
C = 128

NC = 2
NS = 16
NW = NC * NS
EPT = E // NW
IPW = EPT // 128
WCOL = 128
NROWS = N // NS
DUMP = N
NACC = N + 16


def _proj_body(nf, ef, Wn, bn, We, be, h_out, e_out):
    h_out[...] = jax.nn.leaky_relu(
        jnp.dot(nf[...], Wn[...], preferred_element_type=jnp.float32) + bn[...])
    e_out[...] = jax.nn.leaky_relu(
        jnp.dot(ef[...], We[...], preferred_element_type=jnp.float32) + be[...])


def _edge_body(u, v, e, us2, vs2,
               Wau, Wae, Wav, ba_i,
               Wmu, Wme, Wmv, bm_i,
               Weu, Wee, Wev, ben_i,
               wm_out, wden_out, ne_out, vsm_out):
    vsm_out[...] = jnp.where(us2[...] == vs2[...], DUMP, vs2[...])
    uu = u[...]
    vv = v[...]
    ee = e[...]
    dot = functools.partial(jnp.dot, preferred_element_type=jnp.float32)
    att = jax.nn.leaky_relu(
        dot(uu, Wau[...]) + dot(ee, Wae[...]) + dot(vv, Wav[...]) + ba_i[...])
    msg = jax.nn.relu(
        dot(uu, Wmu[...]) + dot(ee, Wme[...]) + dot(vv, Wmv[...]) + bm_i[...])
    ne_out[...] = jax.nn.leaky_relu(
        dot(uu, Weu[...]) + dot(ee, Wee[...]) + dot(vv, Wev[...]) + ben_i[...])
    gmax = jnp.max(att)
    wu = jnp.exp(att - gmax)
    col = lax.broadcasted_iota(jnp.int32, (E, H), 1)
    wm_out[...] = msg * wu
    wden_out[...] = jnp.where(col == 0, wu, 0.0)


def _gru_body(num0, num1, den0, den1, h, Wih_i, bih_i, Whh_i, bhh_i,
              relu_flag, h_out):
    s = num0[...] + num1[...]
    denom = jnp.maximum(den0[:, :1] + den1[:, :1], 1e-30)
    ctx = s / denom
    dot = functools.partial(jnp.dot, preferred_element_type=jnp.float32)
    gi = dot(ctx, Wih_i[...]) + bih_i[...]
    gh = dot(h[...], Whh_i[...]) + bhh_i[...]
    r = jax.nn.sigmoid(gi[:, :H] + gh[:, :H])
    z = jax.nn.sigmoid(gi[:, H:2 * H] + gh[:, H:2 * H])
    n = jnp.tanh(gi[:, 2 * H:] + r * gh[:, 2 * H:])
    new_h = (1.0 - z) * n + z * h[...]
    if relu_flag:
        new_h = jax.nn.relu(new_h)
    h_out[...] = new_h


def _readout_body(h, mask, mat, Wra, bra, Wro, bro, ro_out, a_out):
    hh = h[...]
    t = jnp.tanh(jnp.dot(hh, Wro[...], preferred_element_type=jnp.float32) + bro[...])
    alT = lax.dot_general(Wra[...], hh, (((0,), (1,)), ((), ())),
                          preferred_element_type=jnp.float32)
    alT = jax.nn.leaky_relu(alT + bra[...])
    logits = mask[...] + alT
    rmax = jnp.max(logits, axis=1, keepdims=True)
    ex = jnp.exp(logits - rmax)
    a = ex / jnp.sum(ex, axis=1, keepdims=True) * mat[...]
    a_out[...] = a
    ro_out[...] = jnp.dot(a, t, preferred_element_type=jnp.float32)


def _tc_proj(nf, ef, Wn, bn, We, be):
    return pl.pallas_call(
        _proj_body,
        out_shape=[jax.ShapeDtypeStruct((N, H), jnp.float32),
                   jax.ShapeDtypeStruct((E, HE), jnp.float32)],
    )(nf, ef, Wn, bn, We, be)


def _tc_edge(u, v, e, us2, vs2, ws):
    return pl.pallas_call(
        _edge_body,
        out_shape=[jax.ShapeDtypeStruct((E, WCOL), jnp.float32),
                   jax.ShapeDtypeStruct((E, WCOL), jnp.float32),
                   jax.ShapeDtypeStruct((E, HE), jnp.float32),
                   jax.ShapeDtypeStruct((E // 128, 128), jnp.int32)],
    )(u, v, e, us2, vs2, *ws)


def _tc_gru(num0, num1, den0, den1, h, Wih_i, bih_i, Whh_i, bhh_i, relu_flag):
    def wrapped(n0, n1, d0, d1, hh, a, b, c, d, h_out):
        _gru_body(n0, n1, d0, d1, hh, a, b, c, d, relu_flag, h_out)

    return pl.pallas_call(
        wrapped,
        out_shape=jax.ShapeDtypeStruct((N, H), jnp.float32),
    )(num0, num1, den0, den1, h, Wih_i, bih_i, Whh_i, bhh_i)


def _tc_readout(h, mask, mat, Wra, bra, Wro, bro):
    return pl.pallas_call(
        _readout_body,
        out_shape=[jax.ShapeDtypeStruct((M, H), jnp.float32),
                   jax.ShapeDtypeStruct((M, N), jnp.float32)],
    )(h, mask, mat, Wra, bra, Wro, bro)


def _sc_gather(h, us2, vs2):
    mesh = plsc.VectorSubcoreMesh(core_axis_name="c", subcore_axis_name="s")

    @functools.partial(
        pl.kernel,
        out_type=[jax.ShapeDtypeStruct((E, H), jnp.float32),
                  jax.ShapeDtypeStruct((E, H), jnp.float32)],
        mesh=mesh,
        scratch_types=[pltpu.VMEM((IPW, 128), jnp.int32),
                       pltpu.VMEM((IPW, 128), jnp.int32),
                       pltpu.VMEM((EPT, H), jnp.float32),
                       pltpu.VMEM((EPT, H), jnp.float32),
                       pltpu.SemaphoreType.DMA,
                       pltpu.SemaphoreType.DMA],
    )
    def k(h_hbm, us_hbm, vs_hbm, u_out, v_out, idxu, idxv, rowsu, rowsv,
          semu, semv):
        cid = lax.axis_index("c")
        sid = lax.axis_index("s")
        wid = sid * NC + cid
        base = wid * EPT
        pltpu.sync_copy(us_hbm.at[pl.ds(wid * IPW, IPW)], idxu)
        pltpu.sync_copy(vs_hbm.at[pl.ds(wid * IPW, IPW)], idxv)
        cps = []
        for j in range(IPW):
            cps.append(pltpu.async_copy(
                h_hbm.at[idxu.at[j]], rowsu.at[pl.ds(j * 128, 128)], semu))
            cps.append(pltpu.async_copy(
                h_hbm.at[idxv.at[j]], rowsv.at[pl.ds(j * 128, 128)], semv))
        for cp in cps:
            cp.wait()
        pltpu.sync_copy(rowsu, u_out.at[pl.ds(base, EPT)])
        pltpu.sync_copy(rowsv, v_out.at[pl.ds(base, EPT)])

    return k(h, us2, vs2)


def _sc_scatter(wm, wden, us2, vs2, zer):
    mesh = plsc.VectorSubcoreMesh(core_axis_name="c", subcore_axis_name="s")

    @functools.partial(
        pl.kernel,
        out_type=[jax.ShapeDtypeStruct((N, WCOL), jnp.float32),
                  jax.ShapeDtypeStruct((N, WCOL), jnp.float32),
                  jax.ShapeDtypeStruct((N, WCOL), jnp.float32),
                  jax.ShapeDtypeStruct((N, WCOL), jnp.float32)],
        mesh=mesh,
        scratch_types=[pltpu.VMEM((IPW, 128), jnp.int32),
                       pltpu.VMEM((IPW, 128), jnp.int32),
                       pltpu.VMEM((EPT, WCOL), jnp.float32),
                       pltpu.VMEM((EPT, WCOL), jnp.float32),
                       pltpu.VMEM_SHARED((NACC, WCOL), jnp.float32),
                       pltpu.VMEM_SHARED((NACC, WCOL), jnp.float32)],
    )
    def k(wm_hbm, wden_hbm, us_hbm, vs_hbm, zer_hbm, num_out0, num_out1,
          den_out0, den_out1, idxu, idxv, rows_m, rows_d, acc_n, acc_d):
        cid = lax.axis_index("c")
        sid = lax.axis_index("s")
        wid = sid * NC + cid
        base = wid * EPT
        pltpu.sync_copy(zer_hbm.at[pl.ds(sid * NROWS, NROWS)],
                        acc_n.at[pl.ds(sid * NROWS, NROWS)])
        pltpu.sync_copy(zer_hbm.at[pl.ds(sid * NROWS, NROWS)],
                        acc_d.at[pl.ds(sid * NROWS, NROWS)])
        pltpu.sync_copy(us_hbm.at[pl.ds(wid * IPW, IPW)], idxu)
        pltpu.sync_copy(vs_hbm.at[pl.ds(wid * IPW, IPW)], idxv)
        pltpu.sync_copy(wm_hbm.at[pl.ds(base, EPT)], rows_m)
        pltpu.sync_copy(wden_hbm.at[pl.ds(base, EPT)], rows_d)
        plsc.subcore_barrier()
        for j in range(IPW):
            pltpu.sync_copy(rows_m.at[pl.ds(j * 128, 128)],
                            acc_n.at[idxu.at[j]], add=True)
            pltpu.sync_copy(rows_m.at[pl.ds(j * 128, 128)],
                            acc_n.at[idxv.at[j]], add=True)
            pltpu.sync_copy(rows_d.at[pl.ds(j * 128, 128)],
                            acc_d.at[idxu.at[j]], add=True)
            pltpu.sync_copy(rows_d.at[pl.ds(j * 128, 128)],
                            acc_d.at[idxv.at[j]], add=True)
        plsc.subcore_barrier()

        @pl.when(cid == 0)
        def _():
            pltpu.sync_copy(acc_n.at[pl.ds(sid * NROWS, NROWS)],
                            num_out0.at[pl.ds(sid * NROWS, NROWS)])
            pltpu.sync_copy(acc_d.at[pl.ds(sid * NROWS, NROWS)],
                            den_out0.at[pl.ds(sid * NROWS, NROWS)])

        @pl.when(cid == 1)
        def _():
            pltpu.sync_copy(acc_n.at[pl.ds(sid * NROWS, NROWS)],
                            num_out1.at[pl.ds(sid * NROWS, NROWS)])
            pltpu.sync_copy(acc_d.at[pl.ds(sid * NROWS, NROWS)],
                            den_out1.at[pl.ds(sid * NROWS, NROWS)])

    return k(wm, wden, us2, vs2, zer)


def kernel(node_features, edge_features, us, vs, mol_node_matrix, mol_node_mask,
           node_edge_matrix, node_edge_mask, global_mask, W_n, b_n, W_e, b_e,
           Wm, bm, Wa, ba, Wen, ben, Wih, bih, Whh, bhh, Wra, bra, Wro, bro):
    us2 = us.reshape(E // 128, 128)
    vs2 = vs.reshape(E // 128, 128)
    zer = jnp.zeros((N, WCOL), jnp.float32)

    h, e = _tc_proj(node_features, edge_features,
                    W_n, b_n.reshape(1, H), W_e, b_e.reshape(1, HE))

    for i in range(L):
        ws = (Wa[i][:H], Wa[i][H:H + HE], Wa[i][H + HE:], ba[i].reshape(1, 1),
              Wm[i][:H], Wm[i][H:H + HE], Wm[i][H + HE:], bm[i].reshape(1, C),
              Wen[i][:H], Wen[i][H:H + HE], Wen[i][H + HE:], ben[i].reshape(1, HE))
        u, v = _sc_gather(h, us2, vs2)
        wm, wden, e, vsm = _tc_edge(u, v, e, us2, vs2, ws)
        num0, num1, den0, den1 = _sc_scatter(wm, wden, us2, vsm, zer)
        h = _tc_gru(num0, num1, den0, den1, h, Wih[i], bih[i].reshape(1, 3 * H),
                    Whh[i], bhh[i].reshape(1, 3 * H), relu_flag=(i != L - 1))

    readout, a = _tc_readout(h, mol_node_mask, mol_node_matrix,
                             Wra, bra.reshape(1, 1), Wro, bro.reshape(1, H))
    return readout, a
